# Initial kernel scaffold; baseline (speedup 1.0000x reference)
import functools

import jax
import jax.numpy as jnp
from jax import lax
from jax.experimental import pallas as pl
from jax.experimental.pallas import tpu as pltpu
from jax.experimental.pallas import tpu_sc as plsc

N_NODES = 10000
N_EDGES = 320000
D_IN = 128
D_OUT = 5

_L = 16
_NC = 2
_NS = 16
_NW = _NC * _NS
_CH = 128
_S = -(-N_EDGES // (_NW * _CH))
_EPAD = _NW * _S * _CH
_NPAD = 10240
_ZR = _NPAD // _NS
_CNT_COL = 5


def _proj_body(x_ref, wl_ref, wr_ref, by_ref, bz_ref, y_ref, z_ref):
    xv = x_ref[...]
    y_ref[...] = jnp.dot(xv, wl_ref[...], preferred_element_type=jnp.float32) + by_ref[...]
    z_ref[...] = jnp.dot(xv, wr_ref[...], preferred_element_type=jnp.float32) + bz_ref[...]


def _sc_body(y_hbm, src_hbm, dst_hbm, out_hbm, srcv, dstv, rows, zb, acc, gsem):
    c = lax.axis_index("c")
    s = lax.axis_index("s")
    wid = c * _NS + s

    zero16 = jnp.zeros((_L,), jnp.float32)

    def zstore(i, carry):
        zb[i, :] = zero16
        return carry

    lax.fori_loop(0, _ZR, zstore, 0)
    pltpu.sync_copy(zb, acc.at[pl.ds(s * _ZR, _ZR)])
    plsc.subcore_barrier()

    pltpu.sync_copy(src_hbm.at[wid], srcv)
    pltpu.sync_copy(dst_hbm.at[wid], dstv)

    def estep(j, carry):
        pltpu.async_copy(y_hbm.at[srcv.at[j]], rows, gsem).wait()
        pltpu.sync_copy(rows, acc.at[dstv.at[j]], add=True)
        return carry

    lax.fori_loop(0, _S, estep, 0)
    plsc.subcore_barrier()

    pltpu.sync_copy(acc.at[pl.ds(s * _ZR, _ZR)], out_hbm.at[c, pl.ds(s * _ZR, _ZR)])


def _final_body(agg_ref, z_ref, o_ref):
    a = agg_ref[0] + agg_ref[1]
    cnt = jnp.maximum(a[:, _CNT_COL:_CNT_COL + 1], 1.0)
    o_ref[...] = a / cnt + z_ref[...]


@jax.jit
def kernel(x, edge_index, W_l, b_l, W_r):
    src = edge_index[0].astype(jnp.int32)
    dst = edge_index[1].astype(jnp.int32)
    src = jnp.concatenate([src, jnp.zeros((_EPAD - N_EDGES,), jnp.int32)])
    dst = jnp.concatenate([dst, jnp.full((_EPAD - N_EDGES,), N_NODES, jnp.int32)])
    src_r = src.reshape(_NW, _S, _CH)
    dst_r = dst.reshape(_NW, _S, _CH)

    xp = jnp.pad(x, ((0, _NPAD - N_NODES), (0, 0)))
    wl = jnp.pad(W_l.T, ((0, 0), (0, _L - D_OUT)))
    wr = jnp.pad(W_r.T, ((0, 0), (0, _L - D_OUT)))
    by = jnp.zeros((1, _L), jnp.float32).at[0, _CNT_COL].set(1.0)
    bz = jnp.pad(b_l, (0, _L - D_OUT)).reshape(1, _L)

    y16, z16 = pl.pallas_call(
        _proj_body,
        out_shape=[
            jax.ShapeDtypeStruct((_NPAD, _L), jnp.float32),
            jax.ShapeDtypeStruct((_NPAD, _L), jnp.float32),
        ],
    )(xp, wl, wr, by, bz)

    sc_fn = pl.kernel(
        _sc_body,
        out_type=jax.ShapeDtypeStruct((_NC, _NPAD, _L), jnp.float32),
        mesh=plsc.VectorSubcoreMesh(
            core_axis_name="c", subcore_axis_name="s",
            num_cores=_NC, num_subcores=_NS,
        ),
        scratch_types=[
            pltpu.VMEM((_S, _CH), jnp.int32),
            pltpu.VMEM((_S, _CH), jnp.int32),
            pltpu.VMEM((_CH, _L), jnp.float32),
            pltpu.VMEM((_ZR, _L), jnp.float32),
            pltpu.VMEM_SHARED((_NPAD, _L), jnp.float32),
            pltpu.SemaphoreType.DMA,
        ],
    )
    agg2 = sc_fn(y16, src_r, dst_r)

    out16 = pl.pallas_call(
        _final_body,
        out_shape=jax.ShapeDtypeStruct((_NPAD, _L), jnp.float32),
    )(agg2, z16)

    return out16[:N_NODES, :D_OUT]


# speedup vs baseline: 20.2488x; 20.2488x over previous
import functools

import jax
import jax.numpy as jnp
from jax import lax
from jax.experimental import pallas as pl
from jax.experimental.pallas import tpu as pltpu
from jax.experimental.pallas import tpu_sc as plsc

N_NODES = 10000
N_EDGES = 320000
D_IN = 128
D_OUT = 5

_L = 16
_NC = 2
_NS = 16
_NW = _NC * _NS
_CH = 128
_S = 80
_EPAD = _NW * _S * _CH
_NPAD = 10240
_ZR = _NPAD // _NS
_CNT_COL = 5


def _proj_body(x_ref, wl_ref, wr_ref, by_ref, bz_ref, y_ref, z_ref):
    xv = x_ref[...]
    y_ref[...] = jnp.dot(xv, wl_ref[...], preferred_element_type=jnp.float32) + by_ref[...]
    z_ref[...] = jnp.dot(xv, wr_ref[...], preferred_element_type=jnp.float32) + bz_ref[...]


def _sc_body(y_hbm, src_hbm, dst_hbm, out_hbm, srcv, dstv, rows, ysh, acc,
             gsem):
    c = lax.axis_index("c")
    s = lax.axis_index("s")
    wid = c * _NS + s

    def ystage(k, carry):
        pltpu.sync_copy(y_hbm.at[pl.ds(s * _ZR + k * _CH, _CH)], rows.at[0])
        pltpu.sync_copy(rows.at[0], ysh.at[pl.ds(s * _ZR + k * _CH, _CH)])
        return carry

    lax.fori_loop(0, _ZR // _CH, ystage, 0)

    zero16 = jnp.zeros((_L,), jnp.float32)

    def zrow(i, carry):
        rows[0, i, :] = zero16
        return carry

    lax.fori_loop(0, _CH, zrow, 0)

    def zcopy(k, carry):
        pltpu.sync_copy(rows.at[0], acc.at[pl.ds(s * _ZR + k * _CH, _CH)])
        return carry

    lax.fori_loop(0, _ZR // _CH, zcopy, 0)

    pltpu.sync_copy(src_hbm.at[wid], srcv)
    pltpu.sync_copy(dst_hbm.at[wid], dstv)
    plsc.subcore_barrier()

    def estep(j, carry):
        pltpu.async_copy(ysh.at[srcv.at[j]], rows.at[0], gsem).wait()
        pltpu.sync_copy(rows.at[0], acc.at[dstv.at[j]], add=True)
        return carry

    lax.fori_loop(0, _S, estep, 0)
    plsc.subcore_barrier()

    def ostage(k, carry):
        pltpu.sync_copy(acc.at[pl.ds(s * _ZR + k * _CH, _CH)], rows.at[0])
        pltpu.sync_copy(rows.at[0], out_hbm.at[c, pl.ds(s * _ZR + k * _CH, _CH)])
        return carry

    lax.fori_loop(0, _ZR // _CH, ostage, 0)


def _final_body(agg_ref, z_ref, o_ref):
    a = agg_ref[0] + agg_ref[1]
    cnt = jnp.maximum(a[:, _CNT_COL:_CNT_COL + 1], 1.0)
    o_ref[...] = a / cnt + z_ref[...]


@jax.jit
def kernel(x, edge_index, W_l, b_l, W_r):
    src = edge_index[0].astype(jnp.int32)
    dst = edge_index[1].astype(jnp.int32)
    src = jnp.concatenate([src, jnp.zeros((_EPAD - N_EDGES,), jnp.int32)])
    dst = jnp.concatenate([dst, jnp.full((_EPAD - N_EDGES,), N_NODES, jnp.int32)])
    src_r = src.reshape(_NW, _S, _CH)
    dst_r = dst.reshape(_NW, _S, _CH)

    xp = jnp.pad(x, ((0, _NPAD - N_NODES), (0, 0)))
    wl = jnp.pad(W_l.T, ((0, 0), (0, _L - D_OUT)))
    wr = jnp.pad(W_r.T, ((0, 0), (0, _L - D_OUT)))
    by = jnp.zeros((1, _L), jnp.float32).at[0, _CNT_COL].set(1.0)
    bz = jnp.pad(b_l, (0, _L - D_OUT)).reshape(1, _L)

    y16, z16 = pl.pallas_call(
        _proj_body,
        out_shape=[
            jax.ShapeDtypeStruct((_NPAD, _L), jnp.float32),
            jax.ShapeDtypeStruct((_NPAD, _L), jnp.float32),
        ],
    )(xp, wl, wr, by, bz)

    sc_fn = pl.kernel(
        _sc_body,
        out_type=jax.ShapeDtypeStruct((_NC, _NPAD, _L), jnp.float32),
        mesh=plsc.VectorSubcoreMesh(
            core_axis_name="c", subcore_axis_name="s",
            num_cores=_NC, num_subcores=_NS,
        ),
        compiler_params=pltpu.CompilerParams(use_tc_tiling_on_sc=False),
        scratch_types=[
            pltpu.VMEM((_S, _CH), jnp.int32),
            pltpu.VMEM((_S, _CH), jnp.int32),
            pltpu.VMEM((2, _CH, _L), jnp.float32),
            pltpu.VMEM_SHARED((_NPAD, _L), jnp.float32),
            pltpu.VMEM_SHARED((_NPAD, _L), jnp.float32),
            pltpu.SemaphoreType.DMA,
        ],
    )
    agg2 = sc_fn(y16, src_r, dst_r)

    out16 = pl.pallas_call(
        _final_body,
        out_shape=jax.ShapeDtypeStruct((_NPAD, _L), jnp.float32),
    )(agg2, z16)

    return out16[:N_NODES, :D_OUT]


# speedup vs baseline: 21.3258x; 1.0532x over previous
import functools

import jax
import jax.numpy as jnp
from jax import lax
from jax.experimental import pallas as pl
from jax.experimental.pallas import tpu as pltpu
from jax.experimental.pallas import tpu_sc as plsc

N_NODES = 10000
N_EDGES = 320000
D_IN = 128
D_OUT = 5

_L = 16
_NC = 2
_NS = 16
_NW = _NC * _NS
_CH = 128
_S = 80
_EPAD = _NW * _S * _CH
_NPAD = 10240
_ZR = _NPAD // _NS
_CNT_COL = 5


def _proj_body(x_ref, wl_ref, wr_ref, by_ref, bz_ref, y_ref, z_ref):
    xv = x_ref[...]
    y_ref[...] = jnp.dot(xv, wl_ref[...], preferred_element_type=jnp.float32) + by_ref[...]
    z_ref[...] = jnp.dot(xv, wr_ref[...], preferred_element_type=jnp.float32) + bz_ref[...]


def _sc_body(y_hbm, src_hbm, dst_hbm, out_hbm, srcv, dstv, rows, ysh, acc,
             gs0, gs1, ss0, ss1):
    c = lax.axis_index("c")
    s = lax.axis_index("s")
    wid = c * _NS + s

    def ystage(k, carry):
        pltpu.sync_copy(y_hbm.at[pl.ds(s * _ZR + k * _CH, _CH)], rows.at[0])
        pltpu.sync_copy(rows.at[0], ysh.at[pl.ds(s * _ZR + k * _CH, _CH)])
        return carry

    lax.fori_loop(0, _ZR // _CH, ystage, 0)

    zero16 = jnp.zeros((_L,), jnp.float32)

    def zrow(i, carry):
        rows[0, i, :] = zero16
        return carry

    lax.fori_loop(0, _CH, zrow, 0)

    def zcopy(k, carry):
        pltpu.sync_copy(rows.at[0], acc.at[pl.ds(s * _ZR + k * _CH, _CH)])
        return carry

    lax.fori_loop(0, _ZR // _CH, zcopy, 0)

    pltpu.sync_copy(src_hbm.at[wid], srcv)
    pltpu.sync_copy(dst_hbm.at[wid], dstv)
    plsc.subcore_barrier()

    pltpu.async_copy(ysh.at[srcv.at[0]], rows.at[0], gs0)
    pltpu.async_copy(ysh.at[srcv.at[1]], rows.at[1], gs1)

    def pstep(t, carry):
        j0 = 2 * t
        pltpu.make_async_copy(ysh.at[srcv.at[j0]], rows.at[0], gs0).wait()
        pltpu.async_copy(rows.at[0], acc.at[dstv.at[j0]], ss0, add=True)
        pltpu.make_async_copy(ysh.at[srcv.at[j0 + 1]], rows.at[1], gs1).wait()
        pltpu.async_copy(rows.at[1], acc.at[dstv.at[j0 + 1]], ss1, add=True)

        @pl.when(t + 1 < _S // 2)
        def _():
            pltpu.make_async_copy(rows.at[0], acc.at[dstv.at[j0]], ss0).wait()
            pltpu.async_copy(ysh.at[srcv.at[j0 + 2]], rows.at[0], gs0)
            pltpu.make_async_copy(rows.at[1], acc.at[dstv.at[j0 + 1]], ss1).wait()
            pltpu.async_copy(ysh.at[srcv.at[j0 + 3]], rows.at[1], gs1)

        return carry

    lax.fori_loop(0, _S // 2, pstep, 0)
    pltpu.make_async_copy(rows.at[0], acc.at[dstv.at[_S - 2]], ss0).wait()
    pltpu.make_async_copy(rows.at[1], acc.at[dstv.at[_S - 1]], ss1).wait()
    plsc.subcore_barrier()

    def ostage(k, carry):
        pltpu.sync_copy(acc.at[pl.ds(s * _ZR + k * _CH, _CH)], rows.at[0])
        pltpu.sync_copy(rows.at[0], out_hbm.at[c, pl.ds(s * _ZR + k * _CH, _CH)])
        return carry

    lax.fori_loop(0, _ZR // _CH, ostage, 0)


def _final_body(agg_ref, z_ref, o_ref):
    a = agg_ref[0] + agg_ref[1]
    cnt = jnp.maximum(a[:, _CNT_COL:_CNT_COL + 1], 1.0)
    o_ref[...] = a / cnt + z_ref[...]


@jax.jit
def kernel(x, edge_index, W_l, b_l, W_r):
    src = edge_index[0].astype(jnp.int32)
    dst = edge_index[1].astype(jnp.int32)
    src = jnp.concatenate([src, jnp.zeros((_EPAD - N_EDGES,), jnp.int32)])
    dst = jnp.concatenate([dst, jnp.full((_EPAD - N_EDGES,), N_NODES, jnp.int32)])
    src_r = src.reshape(_NW, _S, _CH)
    dst_r = dst.reshape(_NW, _S, _CH)

    xp = jnp.pad(x, ((0, _NPAD - N_NODES), (0, 0)))
    wl = jnp.pad(W_l.T, ((0, 0), (0, _L - D_OUT)))
    wr = jnp.pad(W_r.T, ((0, 0), (0, _L - D_OUT)))
    by = jnp.zeros((1, _L), jnp.float32).at[0, _CNT_COL].set(1.0)
    bz = jnp.pad(b_l, (0, _L - D_OUT)).reshape(1, _L)

    y16, z16 = pl.pallas_call(
        _proj_body,
        out_shape=[
            jax.ShapeDtypeStruct((_NPAD, _L), jnp.float32),
            jax.ShapeDtypeStruct((_NPAD, _L), jnp.float32),
        ],
    )(xp, wl, wr, by, bz)

    sc_fn = pl.kernel(
        _sc_body,
        out_type=jax.ShapeDtypeStruct((_NC, _NPAD, _L), jnp.float32),
        mesh=plsc.VectorSubcoreMesh(
            core_axis_name="c", subcore_axis_name="s",
            num_cores=_NC, num_subcores=_NS,
        ),
        compiler_params=pltpu.CompilerParams(use_tc_tiling_on_sc=False),
        scratch_types=[
            pltpu.VMEM((_S, _CH), jnp.int32),
            pltpu.VMEM((_S, _CH), jnp.int32),
            pltpu.VMEM((2, _CH, _L), jnp.float32),
            pltpu.VMEM_SHARED((_NPAD, _L), jnp.float32),
            pltpu.VMEM_SHARED((_NPAD, _L), jnp.float32),
            pltpu.SemaphoreType.DMA,
            pltpu.SemaphoreType.DMA,
            pltpu.SemaphoreType.DMA,
            pltpu.SemaphoreType.DMA,
        ],
    )
    agg2 = sc_fn(y16, src_r, dst_r)

    out16 = pl.pallas_call(
        _final_body,
        out_shape=jax.ShapeDtypeStruct((_NPAD, _L), jnp.float32),
    )(agg2, z16)

    return out16[:N_NODES, :D_OUT]


# speedup vs baseline: 22.6851x; 1.0637x over previous
import functools

import jax
import jax.numpy as jnp
from jax import lax
from jax.experimental import pallas as pl
from jax.experimental.pallas import tpu as pltpu
from jax.experimental.pallas import tpu_sc as plsc

N_NODES = 10000
N_EDGES = 320000
D_IN = 128
D_OUT = 5

_L = 16
_NC = 2
_NS = 16
_NW = _NC * _NS
_CH = 128
_S = 80
_EPAD = _NW * _S * _CH
_NPAD = 10240
_ZR = _NPAD // _NS
_CNT_COL = 5


_BM = 1000


def _proj_body(x_ref, wl_ref, wr_ref, bl_ref, y_ref, z_ref):
    xv = x_ref[...]
    dn = (((1,), (1,)), ((), ()))
    y5 = jax.lax.dot_general(xv, wl_ref[...], dn,
                             preferred_element_type=jnp.float32)
    z5 = jax.lax.dot_general(xv, wr_ref[...], dn,
                             preferred_element_type=jnp.float32) + bl_ref[...]
    ones = jnp.ones((_BM, 1), jnp.float32)
    zeros10 = jnp.zeros((_BM, _L - D_OUT - 1), jnp.float32)
    zeros11 = jnp.zeros((_BM, _L - D_OUT), jnp.float32)
    y_ref[...] = jnp.concatenate([y5, ones, zeros10], axis=1)
    z_ref[...] = jnp.concatenate([z5, zeros11], axis=1)


def _sc_body(y_hbm, src_hbm, dst_hbm, out_hbm, srcv, dstv, rows, ysh, acc,
             gs0, gs1, ss0, ss1):
    c = lax.axis_index("c")
    s = lax.axis_index("s")
    wid = c * _NS + s

    def ystage(k, carry):
        pltpu.sync_copy(y_hbm.at[pl.ds(s * _ZR + k * _CH, _CH)], rows.at[0])
        pltpu.sync_copy(rows.at[0], ysh.at[pl.ds(s * _ZR + k * _CH, _CH)])
        return carry

    lax.fori_loop(0, _ZR // _CH, ystage, 0)

    zero16 = jnp.zeros((_L,), jnp.float32)

    def zrow(i, carry):
        rows[0, i, :] = zero16
        return carry

    lax.fori_loop(0, _CH, zrow, 0)

    def zcopy(k, carry):
        pltpu.sync_copy(rows.at[0], acc.at[pl.ds(s * _ZR + k * _CH, _CH)])
        return carry

    lax.fori_loop(0, _ZR // _CH, zcopy, 0)

    pltpu.sync_copy(src_hbm.at[wid], srcv)
    pltpu.sync_copy(dst_hbm.at[wid], dstv)
    plsc.subcore_barrier()

    pltpu.async_copy(ysh.at[srcv.at[0]], rows.at[0], gs0)
    pltpu.async_copy(ysh.at[srcv.at[1]], rows.at[1], gs1)

    def pstep(t, carry):
        j0 = 2 * t
        pltpu.make_async_copy(ysh.at[srcv.at[j0]], rows.at[0], gs0).wait()
        pltpu.async_copy(rows.at[0], acc.at[dstv.at[j0]], ss0, add=True)
        pltpu.make_async_copy(ysh.at[srcv.at[j0 + 1]], rows.at[1], gs1).wait()
        pltpu.async_copy(rows.at[1], acc.at[dstv.at[j0 + 1]], ss1, add=True)

        @pl.when(t + 1 < _S // 2)
        def _():
            pltpu.make_async_copy(rows.at[0], acc.at[dstv.at[j0]], ss0).wait()
            pltpu.async_copy(ysh.at[srcv.at[j0 + 2]], rows.at[0], gs0)
            pltpu.make_async_copy(rows.at[1], acc.at[dstv.at[j0 + 1]], ss1).wait()
            pltpu.async_copy(ysh.at[srcv.at[j0 + 3]], rows.at[1], gs1)

        return carry

    lax.fori_loop(0, _S // 2, pstep, 0)
    pltpu.make_async_copy(rows.at[0], acc.at[dstv.at[_S - 2]], ss0).wait()
    pltpu.make_async_copy(rows.at[1], acc.at[dstv.at[_S - 1]], ss1).wait()
    plsc.subcore_barrier()

    def ostage(k, carry):
        pltpu.sync_copy(acc.at[pl.ds(s * _ZR + k * _CH, _CH)], rows.at[0])
        pltpu.sync_copy(rows.at[0], out_hbm.at[c, pl.ds(s * _ZR + k * _CH, _CH)])
        return carry

    lax.fori_loop(0, _ZR // _CH, ostage, 0)


def _final_body(agg_ref, z_ref, o_ref):
    a = agg_ref[0] + agg_ref[1]
    cnt = jnp.maximum(a[:, _CNT_COL:_CNT_COL + 1], 1.0)
    o_ref[...] = a / cnt + z_ref[...]


@jax.jit
def kernel(x, edge_index, W_l, b_l, W_r):
    src = edge_index[0].astype(jnp.int32)
    dst = edge_index[1].astype(jnp.int32)
    src = jnp.concatenate([src, jnp.zeros((_EPAD - N_EDGES,), jnp.int32)])
    dst = jnp.concatenate([dst, jnp.full((_EPAD - N_EDGES,), N_NODES, jnp.int32)])
    src_r = src.reshape(_NW, _S, _CH)
    dst_r = dst.reshape(_NW, _S, _CH)

    y16, z16 = pl.pallas_call(
        _proj_body,
        grid=(N_NODES // _BM,),
        in_specs=[
            pl.BlockSpec((_BM, D_IN), lambda i: (i, 0)),
            pl.BlockSpec((D_OUT, D_IN), lambda i: (0, 0)),
            pl.BlockSpec((D_OUT, D_IN), lambda i: (0, 0)),
            pl.BlockSpec((D_OUT,), lambda i: (0,)),
        ],
        out_specs=[
            pl.BlockSpec((_BM, _L), lambda i: (i, 0)),
            pl.BlockSpec((_BM, _L), lambda i: (i, 0)),
        ],
        out_shape=[
            jax.ShapeDtypeStruct((_NPAD, _L), jnp.float32),
            jax.ShapeDtypeStruct((_NPAD, _L), jnp.float32),
        ],
    )(x, W_l, W_r, b_l)

    sc_fn = pl.kernel(
        _sc_body,
        out_type=jax.ShapeDtypeStruct((_NC, _NPAD, _L), jnp.float32),
        mesh=plsc.VectorSubcoreMesh(
            core_axis_name="c", subcore_axis_name="s",
            num_cores=_NC, num_subcores=_NS,
        ),
        compiler_params=pltpu.CompilerParams(use_tc_tiling_on_sc=False),
        scratch_types=[
            pltpu.VMEM((_S, _CH), jnp.int32),
            pltpu.VMEM((_S, _CH), jnp.int32),
            pltpu.VMEM((2, _CH, _L), jnp.float32),
            pltpu.VMEM_SHARED((_NPAD, _L), jnp.float32),
            pltpu.VMEM_SHARED((_NPAD, _L), jnp.float32),
            pltpu.SemaphoreType.DMA,
            pltpu.SemaphoreType.DMA,
            pltpu.SemaphoreType.DMA,
            pltpu.SemaphoreType.DMA,
        ],
    )
    agg2 = sc_fn(y16, src_r, dst_r)

    out16 = pl.pallas_call(
        _final_body,
        out_shape=jax.ShapeDtypeStruct((_NPAD, _L), jnp.float32),
    )(agg2, z16)

    return out16[:N_NODES, :D_OUT]


# speedup vs baseline: 26.3439x; 1.1613x over previous
import functools

import jax
import jax.numpy as jnp
from jax import lax
from jax.experimental import pallas as pl
from jax.experimental.pallas import tpu as pltpu
from jax.experimental.pallas import tpu_sc as plsc

N_NODES = 10000
N_EDGES = 320000
D_IN = 128
D_OUT = 5

_L = 16
_NC = 2
_NS = 16
_NW = _NC * _NS
_CH = 128
_ER = N_EDGES // _CH
_SF = _ER // _NW
_XR = _ER - _SF * _NW
_S = _SF + 1
_NPAD = 10240
_ZR = _NPAD // _NS
_CNT_COL = 5


_BM = 2000


def _proj_body(x_ref, wcat_ref, bl_ref, y_ref, z_ref):
    xv = x_ref[...]
    dn = (((1,), (1,)), ((), ()))
    y10 = jax.lax.dot_general(xv, wcat_ref[...], dn,
                              preferred_element_type=jnp.float32)
    ones = jnp.ones((_BM, 1), jnp.float32)
    zeros10 = jnp.zeros((_BM, _L - D_OUT - 1), jnp.float32)
    zeros11 = jnp.zeros((_BM, _L - D_OUT), jnp.float32)
    y_ref[...] = jnp.concatenate([y10[:, 0:D_OUT], ones, zeros10], axis=1)
    z_ref[...] = jnp.concatenate([y10[:, D_OUT:2 * D_OUT] + bl_ref[...], zeros11],
                                 axis=1)


def _sc_body(y_hbm, ev_hbm, out_hbm, srcv, dstv, rows, ysh, acc,
             gs0, gs1, ss0, ss1):
    c = lax.axis_index("c")
    s = lax.axis_index("s")
    wid = c * _NS + s

    def ystage(k, carry):
        pltpu.sync_copy(y_hbm.at[pl.ds(s * _ZR + k * _CH, _CH)], rows.at[0])
        pltpu.sync_copy(rows.at[0], ysh.at[pl.ds(s * _ZR + k * _CH, _CH)])
        return carry

    lax.fori_loop(0, _ZR // _CH, ystage, 0)

    zero16 = jnp.zeros((_L,), jnp.float32)

    def zrow(i, carry):
        rows[0, i, :] = zero16
        return carry

    lax.fori_loop(0, _CH, zrow, 0)

    def zcopy(k, carry):
        pltpu.sync_copy(rows.at[0], acc.at[pl.ds(s * _ZR + k * _CH, _CH)])
        return carry

    lax.fori_loop(0, _ZR // _CH, zcopy, 0)

    pltpu.sync_copy(ev_hbm.at[0, pl.ds(wid * _SF, _SF)], srcv.at[pl.ds(0, _SF)])
    pltpu.sync_copy(ev_hbm.at[1, pl.ds(wid * _SF, _SF)], dstv.at[pl.ds(0, _SF)])

    @pl.when(wid < _XR)
    def _():
        pltpu.sync_copy(ev_hbm.at[0, pl.ds(_SF * _NW + wid, 1)],
                        srcv.at[pl.ds(_SF, 1)])
        pltpu.sync_copy(ev_hbm.at[1, pl.ds(_SF * _NW + wid, 1)],
                        dstv.at[pl.ds(_SF, 1)])

    plsc.subcore_barrier()

    pltpu.async_copy(ysh.at[srcv.at[0]], rows.at[0], gs0)
    pltpu.async_copy(ysh.at[srcv.at[1]], rows.at[1], gs1)

    def pstep(t, carry):
        j0 = 2 * t
        pltpu.make_async_copy(ysh.at[srcv.at[j0]], rows.at[0], gs0).wait()
        pltpu.async_copy(rows.at[0], acc.at[dstv.at[j0]], ss0, add=True)
        pltpu.make_async_copy(ysh.at[srcv.at[j0 + 1]], rows.at[1], gs1).wait()
        pltpu.async_copy(rows.at[1], acc.at[dstv.at[j0 + 1]], ss1, add=True)

        @pl.when(t + 1 < _SF // 2)
        def _():
            pltpu.make_async_copy(rows.at[0], acc.at[dstv.at[j0]], ss0).wait()
            pltpu.async_copy(ysh.at[srcv.at[j0 + 2]], rows.at[0], gs0)
            pltpu.make_async_copy(rows.at[1], acc.at[dstv.at[j0 + 1]], ss1).wait()
            pltpu.async_copy(ysh.at[srcv.at[j0 + 3]], rows.at[1], gs1)

        return carry

    lax.fori_loop(0, _SF // 2, pstep, 0)
    pltpu.make_async_copy(rows.at[0], acc.at[dstv.at[_SF - 2]], ss0).wait()
    pltpu.make_async_copy(rows.at[1], acc.at[dstv.at[_SF - 1]], ss1).wait()

    @pl.when(wid < _XR)
    def _():
        pltpu.async_copy(ysh.at[srcv.at[_SF]], rows.at[0], gs0).wait()
        pltpu.sync_copy(rows.at[0], acc.at[dstv.at[_SF]], add=True)

    plsc.subcore_barrier()

    def ostage(k, carry):
        pltpu.sync_copy(acc.at[pl.ds(s * _ZR + k * _CH, _CH)], rows.at[0])
        pltpu.sync_copy(rows.at[0], out_hbm.at[c, pl.ds(s * _ZR + k * _CH, _CH)])
        return carry

    lax.fori_loop(0, _ZR // _CH, ostage, 0)


def _final_body(agg_ref, z_ref, o_ref):
    a = agg_ref[0] + agg_ref[1]
    cnt = jnp.maximum(a[:, _CNT_COL:_CNT_COL + 1], 1.0)
    o_ref[...] = (a / cnt + z_ref[...])[:, :D_OUT]


@jax.jit
def kernel(x, edge_index, W_l, b_l, W_r):
    ev = edge_index.astype(jnp.int32).reshape(2, _ER, _CH)
    wcat = jnp.concatenate([W_l, W_r], axis=0)

    y16, z16 = pl.pallas_call(
        _proj_body,
        grid=(N_NODES // _BM,),
        in_specs=[
            pl.BlockSpec((_BM, D_IN), lambda i: (i, 0)),
            pl.BlockSpec((2 * D_OUT, D_IN), lambda i: (0, 0)),
            pl.BlockSpec((D_OUT,), lambda i: (0,)),
        ],
        out_specs=[
            pl.BlockSpec((_BM, _L), lambda i: (i, 0)),
            pl.BlockSpec((_BM, _L), lambda i: (i, 0)),
        ],
        out_shape=[
            jax.ShapeDtypeStruct((_NPAD, _L), jnp.float32),
            jax.ShapeDtypeStruct((_NPAD, _L), jnp.float32),
        ],
    )(x, wcat, b_l)

    sc_fn = pl.kernel(
        _sc_body,
        out_type=jax.ShapeDtypeStruct((_NC, _NPAD, _L), jnp.float32),
        mesh=plsc.VectorSubcoreMesh(
            core_axis_name="c", subcore_axis_name="s",
            num_cores=_NC, num_subcores=_NS,
        ),
        compiler_params=pltpu.CompilerParams(use_tc_tiling_on_sc=False),
        scratch_types=[
            pltpu.VMEM((_S, _CH), jnp.int32),
            pltpu.VMEM((_S, _CH), jnp.int32),
            pltpu.VMEM((2, _CH, _L), jnp.float32),
            pltpu.VMEM_SHARED((_NPAD, _L), jnp.float32),
            pltpu.VMEM_SHARED((_NPAD, _L), jnp.float32),
            pltpu.SemaphoreType.DMA,
            pltpu.SemaphoreType.DMA,
            pltpu.SemaphoreType.DMA,
            pltpu.SemaphoreType.DMA,
        ],
    )
    agg2 = sc_fn(y16, ev)

    out = pl.pallas_call(
        _final_body,
        grid=(N_NODES // _BM,),
        in_specs=[
            pl.BlockSpec((_NC, _BM, _L), lambda i: (0, i, 0)),
            pl.BlockSpec((_BM, _L), lambda i: (i, 0)),
        ],
        out_specs=pl.BlockSpec((_BM, D_OUT), lambda i: (i, 0)),
        out_shape=jax.ShapeDtypeStruct((N_NODES, D_OUT), jnp.float32),
    )(agg2, z16)

    return out


# speedup vs baseline: 27.3660x; 1.0388x over previous
import functools

import jax
import jax.numpy as jnp
from jax import lax
from jax.experimental import pallas as pl
from jax.experimental.pallas import tpu as pltpu
from jax.experimental.pallas import tpu_sc as plsc

N_NODES = 10000
N_EDGES = 320000
D_IN = 128
D_OUT = 5

_L = 16
_NC = 2
_NS = 16
_NW = _NC * _NS
_CH = 128
_ER = N_EDGES // _CH
_SF = _ER // _NW
_XR = _ER - _SF * _NW
_S = _SF + 1
_NPAD = 10240
_ZR = _NPAD // _NS
_CNT_COL = 5


_BM = 2000


def _proj_body(x_ref, wcat_ref, bl_ref, y_ref, z_ref):
    xv = x_ref[...]
    dn = (((1,), (1,)), ((), ()))
    y10 = jax.lax.dot_general(xv, wcat_ref[...], dn,
                              preferred_element_type=jnp.float32)
    ones = jnp.ones((_BM, 1), jnp.float32)
    zeros10 = jnp.zeros((_BM, _L - D_OUT - 1), jnp.float32)
    zeros11 = jnp.zeros((_BM, _L - D_OUT), jnp.float32)
    y_ref[...] = jnp.concatenate([y10[:, 0:D_OUT], ones, zeros10], axis=1)
    z_ref[...] = jnp.concatenate([y10[:, D_OUT:2 * D_OUT] + bl_ref[...], zeros11],
                                 axis=1)


def _sc_body(y_hbm, ev_hbm, out_hbm, srcv, dstv, rows, ysh, acc,
             gs0, gs1, ss0, ss1, is0, is1):
    c = lax.axis_index("c")
    s = lax.axis_index("s")
    wid = c * _NS + s

    pltpu.async_copy(ev_hbm.at[0, pl.ds(wid * _SF, _SF)],
                     srcv.at[pl.ds(0, _SF)], is0)
    pltpu.async_copy(ev_hbm.at[1, pl.ds(wid * _SF, _SF)],
                     dstv.at[pl.ds(0, _SF)], is1)

    nk = _ZR // _CH

    def ychunk(k):
        return pl.ds(s * _ZR + k * _CH, _CH)

    pltpu.async_copy(y_hbm.at[ychunk(0)], rows.at[0], ss0)
    for k in range(nk):
        b = k & 1
        if k + 1 < nk:
            if k >= 1:
                pltpu.make_async_copy(rows.at[1 - b], ysh.at[ychunk(k - 1)],
                                      (gs0, gs1)[1 - b]).wait()
            pltpu.async_copy(y_hbm.at[ychunk(k + 1)], rows.at[1 - b],
                             (ss0, ss1)[1 - b])
        pltpu.make_async_copy(y_hbm.at[ychunk(k)], rows.at[b],
                              (ss0, ss1)[b]).wait()
        pltpu.async_copy(rows.at[b], ysh.at[ychunk(k)], (gs0, gs1)[b])
    pltpu.make_async_copy(rows.at[(nk - 1) & 1], ysh.at[ychunk(nk - 1)],
                          (gs0, gs1)[(nk - 1) & 1]).wait()
    pltpu.make_async_copy(rows.at[(nk - 2) & 1], ysh.at[ychunk(nk - 2)],
                          (gs0, gs1)[(nk - 2) & 1]).wait()

    pltpu.make_async_copy(ev_hbm.at[0, pl.ds(wid * _SF, _SF)],
                          srcv.at[pl.ds(0, _SF)], is0).wait()
    pltpu.make_async_copy(ev_hbm.at[1, pl.ds(wid * _SF, _SF)],
                          dstv.at[pl.ds(0, _SF)], is1).wait()

    @pl.when(wid < _XR)
    def _():
        pltpu.sync_copy(ev_hbm.at[0, pl.ds(_SF * _NW + wid, 1)],
                        srcv.at[pl.ds(_SF, 1)])
        pltpu.sync_copy(ev_hbm.at[1, pl.ds(_SF * _NW + wid, 1)],
                        dstv.at[pl.ds(_SF, 1)])

    zero16 = jnp.zeros((_L,), jnp.float32)

    def zrow(i, carry):
        rows[0, i, :] = zero16
        return carry

    lax.fori_loop(0, _CH, zrow, 0)

    def zcopy(k, carry):
        pltpu.sync_copy(rows.at[0], acc.at[pl.ds(s * _ZR + k * _CH, _CH)])
        return carry

    lax.fori_loop(0, _ZR // _CH, zcopy, 0)

    plsc.subcore_barrier()

    pltpu.async_copy(ysh.at[srcv.at[0]], rows.at[0], gs0)
    pltpu.async_copy(ysh.at[srcv.at[1]], rows.at[1], gs1)

    def pstep(t, carry):
        j0 = 2 * t
        pltpu.make_async_copy(ysh.at[srcv.at[j0]], rows.at[0], gs0).wait()
        pltpu.async_copy(rows.at[0], acc.at[dstv.at[j0]], ss0, add=True)
        pltpu.make_async_copy(ysh.at[srcv.at[j0 + 1]], rows.at[1], gs1).wait()
        pltpu.async_copy(rows.at[1], acc.at[dstv.at[j0 + 1]], ss1, add=True)

        @pl.when(t + 1 < _SF // 2)
        def _():
            pltpu.make_async_copy(rows.at[0], acc.at[dstv.at[j0]], ss0).wait()
            pltpu.async_copy(ysh.at[srcv.at[j0 + 2]], rows.at[0], gs0)
            pltpu.make_async_copy(rows.at[1], acc.at[dstv.at[j0 + 1]], ss1).wait()
            pltpu.async_copy(ysh.at[srcv.at[j0 + 3]], rows.at[1], gs1)

        return carry

    lax.fori_loop(0, _SF // 2, pstep, 0)
    pltpu.make_async_copy(rows.at[0], acc.at[dstv.at[_SF - 2]], ss0).wait()
    pltpu.make_async_copy(rows.at[1], acc.at[dstv.at[_SF - 1]], ss1).wait()

    @pl.when(wid < _XR)
    def _():
        pltpu.async_copy(ysh.at[srcv.at[_SF]], rows.at[0], gs0).wait()
        pltpu.sync_copy(rows.at[0], acc.at[dstv.at[_SF]], add=True)

    plsc.subcore_barrier()

    def ostage(k, carry):
        pltpu.sync_copy(acc.at[pl.ds(s * _ZR + k * _CH, _CH)], rows.at[0])
        pltpu.sync_copy(rows.at[0], out_hbm.at[c, pl.ds(s * _ZR + k * _CH, _CH)])
        return carry

    lax.fori_loop(0, _ZR // _CH, ostage, 0)


def _final_body(agg_ref, z_ref, o_ref):
    a = agg_ref[0] + agg_ref[1]
    cnt = jnp.maximum(a[:, _CNT_COL:_CNT_COL + 1], 1.0)
    o_ref[...] = (a / cnt + z_ref[...])[:, :D_OUT]


@jax.jit
def kernel(x, edge_index, W_l, b_l, W_r):
    ev = edge_index.astype(jnp.int32).reshape(2, _ER, _CH)
    wcat = jnp.concatenate([W_l, W_r], axis=0)

    y16, z16 = pl.pallas_call(
        _proj_body,
        grid=(N_NODES // _BM,),
        in_specs=[
            pl.BlockSpec((_BM, D_IN), lambda i: (i, 0)),
            pl.BlockSpec((2 * D_OUT, D_IN), lambda i: (0, 0)),
            pl.BlockSpec((D_OUT,), lambda i: (0,)),
        ],
        out_specs=[
            pl.BlockSpec((_BM, _L), lambda i: (i, 0)),
            pl.BlockSpec((_BM, _L), lambda i: (i, 0)),
        ],
        out_shape=[
            jax.ShapeDtypeStruct((_NPAD, _L), jnp.float32),
            jax.ShapeDtypeStruct((_NPAD, _L), jnp.float32),
        ],
    )(x, wcat, b_l)

    sc_fn = pl.kernel(
        _sc_body,
        out_type=jax.ShapeDtypeStruct((_NC, _NPAD, _L), jnp.float32),
        mesh=plsc.VectorSubcoreMesh(
            core_axis_name="c", subcore_axis_name="s",
            num_cores=_NC, num_subcores=_NS,
        ),
        compiler_params=pltpu.CompilerParams(use_tc_tiling_on_sc=False),
        scratch_types=[
            pltpu.VMEM((_S, _CH), jnp.int32),
            pltpu.VMEM((_S, _CH), jnp.int32),
            pltpu.VMEM((2, _CH, _L), jnp.float32),
            pltpu.VMEM_SHARED((_NPAD, _L), jnp.float32),
            pltpu.VMEM_SHARED((_NPAD, _L), jnp.float32),
            pltpu.SemaphoreType.DMA,
            pltpu.SemaphoreType.DMA,
            pltpu.SemaphoreType.DMA,
            pltpu.SemaphoreType.DMA,
            pltpu.SemaphoreType.DMA,
            pltpu.SemaphoreType.DMA,
        ],
    )
    agg2 = sc_fn(y16, ev)

    out = pl.pallas_call(
        _final_body,
        grid=(N_NODES // _BM,),
        in_specs=[
            pl.BlockSpec((_NC, _BM, _L), lambda i: (0, i, 0)),
            pl.BlockSpec((_BM, _L), lambda i: (i, 0)),
        ],
        out_specs=pl.BlockSpec((_BM, D_OUT), lambda i: (i, 0)),
        out_shape=jax.ShapeDtypeStruct((N_NODES, D_OUT), jnp.float32),
    )(agg2, z16)

    return out
